# baseline (speedup 1.0000x reference)
import functools

import jax
import jax.numpy as jnp
from jax import lax
from jax.experimental import pallas as pl
from jax.experimental.pallas import tpu as pltpu
from jax.experimental.pallas import tpu_sc as plsc

BATCH = 16384
FEAT = 8
NC = 2
NS = 16
NW = NC * NS
B_PER_W = BATCH // NW
IDX_CHUNK = 128
N_CHUNKS = B_PER_W // IDX_CHUNK


def _sc_gather_body(users_hbm, tracks_hbm, user_table_hbm, track_table_hbm,
                    u_out_hbm, t_out_hbm, uidx_v, tidx_v, urows_v, trows_v,
                    sem_idx, sem_rows):
    wid = lax.axis_index("s") * NC + lax.axis_index("c")
    base = wid * B_PER_W
    cp_u = pltpu.make_async_copy(users_hbm.at[pl.ds(base, B_PER_W)], uidx_v,
                                 sem_idx)
    cp_t = pltpu.make_async_copy(tracks_hbm.at[pl.ds(base, B_PER_W)], tidx_v,
                                 sem_idx)
    cp_u.start()
    cp_t.start()
    cp_u.wait()
    cp_t.wait()
    copies = []
    for c in range(N_CHUNKS):
        sl = pl.ds(c * IDX_CHUNK, IDX_CHUNK)
        copies.append(pltpu.make_async_copy(
            user_table_hbm.at[uidx_v.at[sl]], urows_v.at[sl], sem_rows))
        copies.append(pltpu.make_async_copy(
            track_table_hbm.at[tidx_v.at[sl]], trows_v.at[sl], sem_rows))
    for cp in copies:
        cp.start()
    for cp in copies:
        cp.wait()
    pltpu.sync_copy(urows_v, u_out_hbm.at[pl.ds(base, B_PER_W)])
    pltpu.sync_copy(trows_v, t_out_hbm.at[pl.ds(base, B_PER_W)])


@jax.jit
def _sc_gather(users, tracks, user_table, track_table):
    mesh = plsc.VectorSubcoreMesh(core_axis_name="c", subcore_axis_name="s",
                                  num_cores=NC, num_subcores=NS)
    return pl.kernel(
        _sc_gather_body,
        out_type=[
            jax.ShapeDtypeStruct((BATCH, FEAT), jnp.float32),
            jax.ShapeDtypeStruct((BATCH, FEAT), jnp.float32),
        ],
        mesh=mesh,
        compiler_params=pltpu.CompilerParams(use_tc_tiling_on_sc=False),
        scratch_types=[
            pltpu.VMEM((B_PER_W,), jnp.int32),
            pltpu.VMEM((B_PER_W,), jnp.int32),
            pltpu.VMEM((B_PER_W, FEAT), jnp.float32),
            pltpu.VMEM((B_PER_W, FEAT), jnp.float32),
            pltpu.SemaphoreType.DMA,
            pltpu.SemaphoreType.DMA,
        ],
    )(users, tracks, user_table, track_table)


BLK = 2048


def _mlp_body(u_ref, t_ref, w1a_ref, w1b_ref, b1_ref, w2_ref, b2_ref,
              w3_ref, b3_ref, out_ref):
    h = jnp.dot(u_ref[...], w1a_ref[...], preferred_element_type=jnp.float32)
    h += jnp.dot(t_ref[...], w1b_ref[...], preferred_element_type=jnp.float32)
    h = jax.nn.relu(h + b1_ref[...])
    h = jnp.dot(h, w2_ref[...], preferred_element_type=jnp.float32)
    h = jax.nn.relu(h + b2_ref[...])
    o = jnp.dot(h, w3_ref[...], preferred_element_type=jnp.float32)
    out_ref[...] = jax.nn.sigmoid(o + b3_ref[...])


@jax.jit
def _mlp(u_e, t_e, W1, b1, W2, b2, W3, b3):
    w1a, w1b = W1[:FEAT], W1[FEAT:]
    grid = BATCH // BLK
    return pl.pallas_call(
        _mlp_body,
        grid=(grid,),
        in_specs=[
            pl.BlockSpec((BLK, FEAT), lambda i: (i, 0)),
            pl.BlockSpec((BLK, FEAT), lambda i: (i, 0)),
            pl.BlockSpec((FEAT, 64), lambda i: (0, 0)),
            pl.BlockSpec((FEAT, 64), lambda i: (0, 0)),
            pl.BlockSpec((1, 64), lambda i: (0, 0)),
            pl.BlockSpec((64, 32), lambda i: (0, 0)),
            pl.BlockSpec((1, 32), lambda i: (0, 0)),
            pl.BlockSpec((32, 1), lambda i: (0, 0)),
            pl.BlockSpec((1, 1), lambda i: (0, 0)),
        ],
        out_specs=pl.BlockSpec((BLK, 1), lambda i: (i, 0)),
        out_shape=jax.ShapeDtypeStruct((BATCH, 1), jnp.float32),
    )(u_e, t_e, w1a, w1b, b1.reshape(1, 64), W2, b2.reshape(1, 32),
      W3, b3.reshape(1, 1))


def kernel(users, tracks, user_table, track_table, W1, b1, W2, b2, W3, b3):
    u_e, t_e = _sc_gather(users, tracks, user_table, track_table)
    return _mlp(u_e, t_e, W1, b1, W2, b2, W3, b3)


# speedup vs baseline: 7.7003x; 7.7003x over previous
import jax
import jax.numpy as jnp
from jax import lax
from jax.experimental import pallas as pl
from jax.experimental.pallas import tpu as pltpu
from jax.experimental.pallas import tpu_sc as plsc

BATCH = 16384
FEAT = 8
NC = 2
NS = 16
NW = NC * NS
B_PER_W = BATCH // NW
LANES = 16
N_GROUPS = B_PER_W // LANES
TILE = 128


def _sc_gather_body(users_hbm, tracks_hbm, utbl_hbm, ttbl_hbm,
                    u_out_hbm, t_out_hbm, uidx_v, tidx_v,
                    utiles_v, ttiles_v, urows_v, trows_v, sem_idx, sem_data):
    wid = lax.axis_index("s") * NC + lax.axis_index("c")
    base = wid * B_PER_W
    cp_u = pltpu.make_async_copy(users_hbm.at[pl.ds(base, B_PER_W)], uidx_v,
                                 sem_idx)
    cp_t = pltpu.make_async_copy(tracks_hbm.at[pl.ds(base, B_PER_W)], tidx_v,
                                 sem_idx)
    cp_u.start()
    cp_t.start()
    cp_u.wait()
    cp_t.wait()

    lanes = lax.iota(jnp.int32, LANES)

    def starts_of(vec):
        return vec & ~jnp.int32(TILE - 1)

    def loop(g, carry):
        uvec = uidx_v[pl.ds(g * LANES, LANES)]
        tvec = tidx_v[pl.ds(g * LANES, LANES)]
        us = starts_of(uvec)
        ts = starts_of(tvec)
        for j in range(LANES):
            su = pl.multiple_of(us[j], TILE)
            st = pl.multiple_of(ts[j], TILE)
            pltpu.make_async_copy(utbl_hbm.at[:, pl.ds(su, TILE)],
                                  utiles_v.at[j], sem_data).start()
            pltpu.make_async_copy(ttbl_hbm.at[:, pl.ds(st, TILE)],
                                  ttiles_v.at[j], sem_data).start()
        pltpu.make_async_copy(utbl_hbm.at[:, pl.ds(0, TILE * LANES)],
                              utiles_v, sem_data).wait()
        pltpu.make_async_copy(ttbl_hbm.at[:, pl.ds(0, TILE * LANES)],
                              ttiles_v, sem_data).wait()
        ulane = uvec - us
        tlane = tvec - ts
        for f in range(FEAT):
            fvec = jnp.full((LANES,), f, jnp.int32)
            uvals = plsc.load_gather(utiles_v, [lanes, fvec, ulane])
            tvals = plsc.load_gather(ttiles_v, [lanes, fvec, tlane])
            urows_v[pl.ds(f * B_PER_W + g * LANES, LANES)] = uvals
            trows_v[pl.ds(f * B_PER_W + g * LANES, LANES)] = tvals
        return carry

    lax.fori_loop(0, N_GROUPS, loop, 0)

    for f in range(FEAT):
        pltpu.sync_copy(
            urows_v.at[pl.ds(f * B_PER_W, B_PER_W)],
            u_out_hbm.at[pl.ds(f * BATCH + base, B_PER_W)])
        pltpu.sync_copy(
            trows_v.at[pl.ds(f * B_PER_W, B_PER_W)],
            t_out_hbm.at[pl.ds(f * BATCH + base, B_PER_W)])


def _sc_gather(users, tracks, utbl_t, ttbl_t):
    mesh = plsc.VectorSubcoreMesh(core_axis_name="c", subcore_axis_name="s",
                                  num_cores=NC, num_subcores=NS)
    return pl.kernel(
        _sc_gather_body,
        out_type=[
            jax.ShapeDtypeStruct((FEAT * BATCH,), jnp.float32),
            jax.ShapeDtypeStruct((FEAT * BATCH,), jnp.float32),
        ],
        mesh=mesh,
        compiler_params=pltpu.CompilerParams(needs_layout_passes=False),
        scratch_types=[
            pltpu.VMEM((B_PER_W,), jnp.int32),
            pltpu.VMEM((B_PER_W,), jnp.int32),
            pltpu.VMEM((LANES, FEAT, TILE), jnp.float32),
            pltpu.VMEM((LANES, FEAT, TILE), jnp.float32),
            pltpu.VMEM((FEAT * B_PER_W,), jnp.float32),
            pltpu.VMEM((FEAT * B_PER_W,), jnp.float32),
            pltpu.SemaphoreType.DMA,
            pltpu.SemaphoreType.DMA,
        ],
    )(users, tracks, utbl_t, ttbl_t)


BLK = 2048


def _mlp_body(u_ref, t_ref, w1a_ref, w1b_ref, b1_ref, w2_ref, b2_ref,
              w3_ref, b3_ref, out_ref):
    cdims = (((0,), (0,)), ((), ()))
    h = lax.dot_general(u_ref[...], w1a_ref[...], cdims,
                        preferred_element_type=jnp.float32)
    h += lax.dot_general(t_ref[...], w1b_ref[...], cdims,
                         preferred_element_type=jnp.float32)
    h = jax.nn.relu(h + b1_ref[...])
    h = jnp.dot(h, w2_ref[...], preferred_element_type=jnp.float32)
    h = jax.nn.relu(h + b2_ref[...])
    o = jnp.dot(h, w3_ref[...], preferred_element_type=jnp.float32)
    out_ref[...] = jax.nn.sigmoid(o + b3_ref[...])


def _mlp(u_et, t_et, W1, b1, W2, b2, W3, b3):
    w1a, w1b = W1[:FEAT], W1[FEAT:]
    grid = BATCH // BLK
    return pl.pallas_call(
        _mlp_body,
        grid=(grid,),
        in_specs=[
            pl.BlockSpec((FEAT, BLK), lambda i: (0, i)),
            pl.BlockSpec((FEAT, BLK), lambda i: (0, i)),
            pl.BlockSpec((FEAT, 64), lambda i: (0, 0)),
            pl.BlockSpec((FEAT, 64), lambda i: (0, 0)),
            pl.BlockSpec((1, 64), lambda i: (0, 0)),
            pl.BlockSpec((64, 32), lambda i: (0, 0)),
            pl.BlockSpec((1, 32), lambda i: (0, 0)),
            pl.BlockSpec((32, 1), lambda i: (0, 0)),
            pl.BlockSpec((1, 1), lambda i: (0, 0)),
        ],
        out_specs=pl.BlockSpec((BLK, 1), lambda i: (i, 0)),
        out_shape=jax.ShapeDtypeStruct((BATCH, 1), jnp.float32),
    )(u_et, t_et, w1a, w1b, b1.reshape(1, 64), W2, b2.reshape(1, 32),
      W3, b3.reshape(1, 1))


def kernel(users, tracks, user_table, track_table, W1, b1, W2, b2, W3, b3):
    u_flat, t_flat = _sc_gather(users, tracks, user_table.T, track_table.T)
    u_et = u_flat.reshape(FEAT, BATCH)
    t_et = t_flat.reshape(FEAT, BATCH)
    return _mlp(u_et, t_et, W1, b1, W2, b2, W3, b3)


# speedup vs baseline: 7.9410x; 1.0313x over previous
import jax
import jax.numpy as jnp
from jax import lax
from jax.experimental import pallas as pl
from jax.experimental.pallas import tpu as pltpu
from jax.experimental.pallas import tpu_sc as plsc

BATCH = 16384
FEAT = 8
NC = 2
NS = 16
NW = NC * NS
B_PER_W = BATCH // NW
LANES = 16
N_GROUPS = B_PER_W // LANES
TILE = 128


def _sc_gather_body(users_hbm, tracks_hbm, utbl_hbm, ttbl_hbm,
                    u_out_hbm, t_out_hbm, uidx_v, tidx_v,
                    utiles_v, ttiles_v, urows_v, trows_v, sem_idx, sem_data):
    wid = lax.axis_index("s") * NC + lax.axis_index("c")
    base = wid * B_PER_W
    cp_u = pltpu.make_async_copy(users_hbm.at[pl.ds(base, B_PER_W)], uidx_v,
                                 sem_idx)
    cp_t = pltpu.make_async_copy(tracks_hbm.at[pl.ds(base, B_PER_W)], tidx_v,
                                 sem_idx)
    cp_u.start()
    cp_t.start()
    cp_u.wait()
    cp_t.wait()

    lanes = lax.iota(jnp.int32, LANES)

    def starts_of(vec):
        return vec & ~jnp.int32(TILE - 1)

    def loop(g, carry):
        uvec = uidx_v[pl.ds(g * LANES, LANES)]
        tvec = tidx_v[pl.ds(g * LANES, LANES)]
        us = starts_of(uvec)
        ts = starts_of(tvec)
        for j in range(LANES):
            su = pl.multiple_of(us[j], TILE)
            st = pl.multiple_of(ts[j], TILE)
            pltpu.make_async_copy(utbl_hbm.at[:, pl.ds(su, TILE)],
                                  utiles_v.at[j], sem_data).start()
            pltpu.make_async_copy(ttbl_hbm.at[:, pl.ds(st, TILE)],
                                  ttiles_v.at[j], sem_data).start()
        pltpu.make_async_copy(utbl_hbm.at[:, pl.ds(0, TILE * LANES)],
                              utiles_v, sem_data).wait()
        pltpu.make_async_copy(ttbl_hbm.at[:, pl.ds(0, TILE * LANES)],
                              ttiles_v, sem_data).wait()
        ulane = uvec - us
        tlane = tvec - ts
        for f in range(FEAT):
            fvec = jnp.full((LANES,), f, jnp.int32)
            uvals = plsc.load_gather(utiles_v, [lanes, fvec, ulane])
            tvals = plsc.load_gather(ttiles_v, [lanes, fvec, tlane])
            urows_v[pl.ds(f * B_PER_W + g * LANES, LANES)] = uvals
            trows_v[pl.ds(f * B_PER_W + g * LANES, LANES)] = tvals
        return carry

    lax.fori_loop(0, N_GROUPS, loop, 0)

    for f in range(FEAT):
        pltpu.sync_copy(
            urows_v.at[pl.ds(f * B_PER_W, B_PER_W)],
            u_out_hbm.at[f, pl.ds(base, B_PER_W)])
        pltpu.sync_copy(
            trows_v.at[pl.ds(f * B_PER_W, B_PER_W)],
            t_out_hbm.at[f, pl.ds(base, B_PER_W)])


def _sc_gather(users, tracks, utbl_t, ttbl_t):
    mesh = plsc.VectorSubcoreMesh(core_axis_name="c", subcore_axis_name="s",
                                  num_cores=NC, num_subcores=NS)
    return pl.kernel(
        _sc_gather_body,
        out_type=[
            jax.ShapeDtypeStruct((FEAT, BATCH), jnp.float32),
            jax.ShapeDtypeStruct((FEAT, BATCH), jnp.float32),
        ],
        mesh=mesh,
        compiler_params=pltpu.CompilerParams(needs_layout_passes=False),
        scratch_types=[
            pltpu.VMEM((B_PER_W,), jnp.int32),
            pltpu.VMEM((B_PER_W,), jnp.int32),
            pltpu.VMEM((LANES, FEAT, TILE), jnp.float32),
            pltpu.VMEM((LANES, FEAT, TILE), jnp.float32),
            pltpu.VMEM((FEAT * B_PER_W,), jnp.float32),
            pltpu.VMEM((FEAT * B_PER_W,), jnp.float32),
            pltpu.SemaphoreType.DMA,
            pltpu.SemaphoreType.DMA,
        ],
    )(users, tracks, utbl_t, ttbl_t)


BLK = 2048


def _mlp_body(u_ref, t_ref, w1a_ref, w1b_ref, b1_ref, w2_ref, b2_ref,
              w3_ref, b3_ref, out_ref):
    cdims = (((0,), (0,)), ((), ()))
    h = lax.dot_general(u_ref[...], w1a_ref[...], cdims,
                        preferred_element_type=jnp.float32)
    h += lax.dot_general(t_ref[...], w1b_ref[...], cdims,
                         preferred_element_type=jnp.float32)
    h = jax.nn.relu(h + b1_ref[...])
    h = jnp.dot(h, w2_ref[...], preferred_element_type=jnp.float32)
    h = jax.nn.relu(h + b2_ref[...])
    o = jnp.dot(h, w3_ref[...], preferred_element_type=jnp.float32)
    out_ref[...] = jax.nn.sigmoid(o + b3_ref[...])


def _mlp(u_et, t_et, W1, b1, W2, b2, W3, b3):
    w1a, w1b = W1[:FEAT], W1[FEAT:]
    grid = BATCH // BLK
    return pl.pallas_call(
        _mlp_body,
        grid=(grid,),
        in_specs=[
            pl.BlockSpec((FEAT, BLK), lambda i: (0, i)),
            pl.BlockSpec((FEAT, BLK), lambda i: (0, i)),
            pl.BlockSpec((FEAT, 64), lambda i: (0, 0)),
            pl.BlockSpec((FEAT, 64), lambda i: (0, 0)),
            pl.BlockSpec((1, 64), lambda i: (0, 0)),
            pl.BlockSpec((64, 32), lambda i: (0, 0)),
            pl.BlockSpec((1, 32), lambda i: (0, 0)),
            pl.BlockSpec((32, 1), lambda i: (0, 0)),
            pl.BlockSpec((1, 1), lambda i: (0, 0)),
        ],
        out_specs=pl.BlockSpec((BLK, 1), lambda i: (i, 0)),
        out_shape=jax.ShapeDtypeStruct((BATCH, 1), jnp.float32),
    )(u_et, t_et, w1a, w1b, b1.reshape(1, 64), W2, b2.reshape(1, 32),
      W3, b3.reshape(1, 1))


def kernel(users, tracks, user_table, track_table, W1, b1, W2, b2, W3, b3):
    u_et, t_et = _sc_gather(users, tracks, user_table.T, track_table.T)
    return _mlp(u_et, t_et, W1, b1, W2, b2, W3, b3)
